# baseline (speedup 1.0000x reference)
import functools

import jax
import jax.numpy as jnp
from jax import lax
from jax.experimental import pallas as pl
from jax.experimental.pallas import tpu as pltpu
from jax.experimental.pallas import tpu_sc as plsc

N = 10000
E = 320000
F = 128

NC = 2
NS = 16
L = 16
NW = NC * NS
FPT = F // NW
RPW = FPT * N
CHUNK = 4000
NCHUNK = E // CHUNK
NWPC = RPW // CHUNK


def _spmm_body(readout, a_hbm, u_hbm, src_hbm, dst_hbm, w_hbm, wr_hbm,
               out_hbm, a_v, z_v, sb0, sb1, db0, db1, wb0, wb1,
               sem0, sem1, semA, semU):
    wid = lax.axis_index("s") * NC + lax.axis_index("c")
    base = wid * RPW

    acp = pltpu.async_copy(a_hbm.at[pl.ds(base, RPW)], a_v, semA)
    ucp = pltpu.async_copy(u_hbm.at[pl.ds(base, RPW)], z_v, semU)

    sbs = (sb0, sb1)
    dbs = (db0, db1)
    wbs = (wb0, wb1)
    sems = (sem0, sem1)

    def _issue(c, b):
        off = c * CHUNK
        pltpu.async_copy(src_hbm.at[pl.ds(off, CHUNK)], sbs[b], sems[b])
        pltpu.async_copy(dst_hbm.at[pl.ds(off, CHUNK)], dbs[b], sems[b])
        pltpu.async_copy(w_hbm.at[pl.ds(off, CHUNK)], wbs[b], sems[b])

    def _drain(c, b):
        off = c * CHUNK
        pltpu.make_async_copy(src_hbm.at[pl.ds(off, CHUNK)], sbs[b],
                              sems[b]).wait()
        pltpu.make_async_copy(dst_hbm.at[pl.ds(off, CHUNK)], dbs[b],
                              sems[b]).wait()
        pltpu.make_async_copy(w_hbm.at[pl.ds(off, CHUNK)], wbs[b],
                              sems[b]).wait()

    _issue(0, 0)
    _issue(1, 1)
    acp.wait()
    ucp.wait()

    def _pair_body(cp, carry):
        c0 = cp * 2
        for b in range(2):
            c = c0 + b
            _drain(c, b)

            @plsc.parallel_loop(0, CHUNK // L, unroll=4)
            def _vec_body(j):
                s = sbs[b][pl.ds(j * L, L)]
                d = dbs[b][pl.ds(j * L, L)]
                ww = wbs[b][pl.ds(j * L, L)]
                for f in range(FPT):
                    g = plsc.load_gather(a_v, [s + (f * N)])
                    plsc.addupdate_scatter(z_v, [d + (f * N)], g * ww)

            @pl.when(c + 2 < NCHUNK)
            def _refill():
                _issue(c + 2, b)

        return carry

    lax.fori_loop(0, NCHUNK // 2, _pair_body, 0)

    if not readout:
        @plsc.parallel_loop(0, RPW // L, unroll=8)
        def _relu_body(j):
            z_v[pl.ds(j * L, L)] = jnp.maximum(z_v[pl.ds(j * L, L)], 0.0)

        pltpu.sync_copy(z_v, out_hbm.at[pl.ds(base, RPW)])
    else:
        pltpu.async_copy(wr_hbm.at[pl.ds(base, CHUNK)], wb0, sem0)
        pltpu.async_copy(wr_hbm.at[pl.ds(base + CHUNK, CHUNK)], wb1, sem1)

        def _piece_body(p, acc):
            for b in range(2):
                pc = p * 2 + b
                off = pc * CHUNK
                pltpu.make_async_copy(
                    wr_hbm.at[pl.ds(base + off, CHUNK)], wbs[b], sems[b]
                ).wait()

                def _dot_body(j, acc2):
                    y = jnp.maximum(z_v[pl.ds(off + j * L, L)], 0.0)
                    w = wbs[b][pl.ds(j * L, L)]
                    return acc2 + y * w

                acc = lax.fori_loop(0, CHUNK // L, _dot_body, acc, unroll=4)

                @pl.when(pc + 2 < NWPC)
                def _refill_w():
                    pltpu.async_copy(
                        wr_hbm.at[pl.ds(base + off + 2 * CHUNK, CHUNK)],
                        wbs[b], sems[b])

            return acc

        acc = lax.fori_loop(0, NWPC // 2, _piece_body,
                            jnp.zeros((L,), jnp.float32))
        a_v[pl.ds(0, L)] = acc
        pltpu.sync_copy(a_v.at[pl.ds(0, L)], out_hbm.at[pl.ds(wid * L, L)])


def _spmm_sc(readout, a_flat, u_flat, src, dst, w, w_readout):
    out_type = (jax.ShapeDtypeStruct((NW * L,), jnp.float32) if readout
                else jax.ShapeDtypeStruct((F * N,), jnp.float32))
    return pl.kernel(
        functools.partial(_spmm_body, readout),
        mesh=plsc.VectorSubcoreMesh(core_axis_name="c", subcore_axis_name="s"),
        compiler_params=pltpu.CompilerParams(needs_layout_passes=False),
        out_type=out_type,
        scratch_types=[
            pltpu.VMEM((RPW,), jnp.float32),
            pltpu.VMEM((RPW,), jnp.float32),
            pltpu.VMEM((CHUNK,), jnp.int32),
            pltpu.VMEM((CHUNK,), jnp.int32),
            pltpu.VMEM((CHUNK,), jnp.int32),
            pltpu.VMEM((CHUNK,), jnp.int32),
            pltpu.VMEM((CHUNK,), jnp.float32),
            pltpu.VMEM((CHUNK,), jnp.float32),
            pltpu.SemaphoreType.DMA,
            pltpu.SemaphoreType.DMA,
            pltpu.SemaphoreType.DMA,
            pltpu.SemaphoreType.DMA,
        ],
    )(a_flat, u_flat, src, dst, w, w_readout)


def _tc_pre_kernel(x_ref, h0_ref, h1_ref, b_ref, u_ref, a_ref):
    x = x_ref[...]
    u_ref[...] = jnp.dot(h0_ref[...], x,
                         preferred_element_type=jnp.float32) + b_ref[...]
    a_ref[...] = jnp.dot(h1_ref[...], x, preferred_element_type=jnp.float32)


def _tc_pre(x, h0, h1, b):
    return pl.pallas_call(
        _tc_pre_kernel,
        out_shape=[
            jax.ShapeDtypeStruct((F, N), jnp.float32),
            jax.ShapeDtypeStruct((F, N), jnp.float32),
        ],
    )(x, h0, h1, b)


def kernel(x, edge_index_clique, edge_weight_clique, edge_index_line,
           edge_weight_line, h_clique, b_clique, h_line, b_line, W_mlp, b_mlp):
    X = x[0]

    h0c = h_clique[:, 0, 0, :]
    h1c = h_clique[:, 0, 1, :]
    h0l = h_line[:, 0, 0, :]
    h1l = h_line[:, 0, 1, :]

    src_c = edge_index_clique[1].astype(jnp.int32)
    dst_c = edge_index_clique[0].astype(jnp.int32)
    src_l = edge_index_line[1].astype(jnp.int32)
    dst_l = edge_index_line[0].astype(jnp.int32)
    w_flat = W_mlp.reshape(F * N)

    u1, a1 = _tc_pre(X, h0c, h1c, b_clique)
    y1 = _spmm_sc(False, a1.reshape(F * N), u1.reshape(F * N),
                  src_c, dst_c, edge_weight_clique, w_flat)
    u2, a2 = _tc_pre(y1.reshape(F, N), h0l, h1l, b_line)
    parts = _spmm_sc(True, a2.reshape(F * N), u2.reshape(F * N),
                     src_l, dst_l, edge_weight_line, w_flat)
    return (jnp.sum(parts) + b_mlp[0]).reshape(1, 1)


# speedup vs baseline: 1.0090x; 1.0090x over previous
import functools

import jax
import jax.numpy as jnp
from jax import lax
from jax.experimental import pallas as pl
from jax.experimental.pallas import tpu as pltpu
from jax.experimental.pallas import tpu_sc as plsc

N = 10000
E = 320000
F = 128

NC = 2
NS = 16
L = 16
NW = NC * NS
FPT = F // NW
RPW = FPT * N
CHUNK = 4000
NCHUNK = E // CHUNK
NWPC = RPW // CHUNK


def _spmm_body(readout, a_hbm, u_hbm, src_hbm, dst_hbm, w_hbm, wr_hbm,
               out_hbm, a_v, z_v, sb0, sb1, db0, db1, wb0, wb1,
               sem0, sem1, semA, semU):
    wid = lax.axis_index("s") * NC + lax.axis_index("c")
    base = wid * RPW

    acp = pltpu.async_copy(a_hbm.at[pl.ds(base, RPW)], a_v, semA)
    ucp = pltpu.async_copy(u_hbm.at[pl.ds(base, RPW)], z_v, semU)

    sbs = (sb0, sb1)
    dbs = (db0, db1)
    wbs = (wb0, wb1)
    sems = (sem0, sem1)

    def _issue(c, b):
        off = c * CHUNK
        pltpu.async_copy(src_hbm.at[pl.ds(off, CHUNK)], sbs[b], sems[b])
        pltpu.async_copy(dst_hbm.at[pl.ds(off, CHUNK)], dbs[b], sems[b])
        pltpu.async_copy(w_hbm.at[pl.ds(off, CHUNK)], wbs[b], sems[b])

    def _drain(c, b):
        off = c * CHUNK
        pltpu.make_async_copy(src_hbm.at[pl.ds(off, CHUNK)], sbs[b],
                              sems[b]).wait()
        pltpu.make_async_copy(dst_hbm.at[pl.ds(off, CHUNK)], dbs[b],
                              sems[b]).wait()
        pltpu.make_async_copy(w_hbm.at[pl.ds(off, CHUNK)], wbs[b],
                              sems[b]).wait()

    _issue(0, 0)
    _issue(1, 1)
    acp.wait()
    ucp.wait()

    def _pair_body(cp, carry):
        c0 = cp * 2
        for b in range(2):
            c = c0 + b
            _drain(c, b)

            @plsc.parallel_loop(0, CHUNK // L, unroll=5)
            def _vec_body(j):
                s = sbs[b][pl.ds(j * L, L)]
                d = dbs[b][pl.ds(j * L, L)]
                ww = wbs[b][pl.ds(j * L, L)]
                for f in range(FPT):
                    g = plsc.load_gather(a_v, [s + (f * N)])
                    plsc.addupdate_scatter(z_v, [d + (f * N)], g * ww)

            @pl.when(c + 2 < NCHUNK)
            def _refill():
                _issue(c + 2, b)

        return carry

    lax.fori_loop(0, NCHUNK // 2, _pair_body, 0)

    if not readout:
        @plsc.parallel_loop(0, RPW // L, unroll=8)
        def _relu_body(j):
            z_v[pl.ds(j * L, L)] = jnp.maximum(z_v[pl.ds(j * L, L)], 0.0)

        pltpu.sync_copy(z_v, out_hbm.at[pl.ds(base, RPW)])
    else:
        pltpu.async_copy(wr_hbm.at[pl.ds(base, CHUNK)], wb0, sem0)
        pltpu.async_copy(wr_hbm.at[pl.ds(base + CHUNK, CHUNK)], wb1, sem1)

        def _piece_body(p, acc):
            for b in range(2):
                pc = p * 2 + b
                off = pc * CHUNK
                pltpu.make_async_copy(
                    wr_hbm.at[pl.ds(base + off, CHUNK)], wbs[b], sems[b]
                ).wait()

                def _dot_body(j, acc2):
                    y = jnp.maximum(z_v[pl.ds(off + j * L, L)], 0.0)
                    w = wbs[b][pl.ds(j * L, L)]
                    return acc2 + y * w

                acc = lax.fori_loop(0, CHUNK // L, _dot_body, acc, unroll=4)

                @pl.when(pc + 2 < NWPC)
                def _refill_w():
                    pltpu.async_copy(
                        wr_hbm.at[pl.ds(base + off + 2 * CHUNK, CHUNK)],
                        wbs[b], sems[b])

            return acc

        acc = lax.fori_loop(0, NWPC // 2, _piece_body,
                            jnp.zeros((L,), jnp.float32))
        a_v[pl.ds(0, L)] = acc
        pltpu.sync_copy(a_v.at[pl.ds(0, L)], out_hbm.at[pl.ds(wid * L, L)])


def _spmm_sc(readout, a_flat, u_flat, src, dst, w, w_readout):
    out_type = (jax.ShapeDtypeStruct((NW * L,), jnp.float32) if readout
                else jax.ShapeDtypeStruct((F * N,), jnp.float32))
    return pl.kernel(
        functools.partial(_spmm_body, readout),
        mesh=plsc.VectorSubcoreMesh(core_axis_name="c", subcore_axis_name="s"),
        compiler_params=pltpu.CompilerParams(needs_layout_passes=False),
        out_type=out_type,
        scratch_types=[
            pltpu.VMEM((RPW,), jnp.float32),
            pltpu.VMEM((RPW,), jnp.float32),
            pltpu.VMEM((CHUNK,), jnp.int32),
            pltpu.VMEM((CHUNK,), jnp.int32),
            pltpu.VMEM((CHUNK,), jnp.int32),
            pltpu.VMEM((CHUNK,), jnp.int32),
            pltpu.VMEM((CHUNK,), jnp.float32),
            pltpu.VMEM((CHUNK,), jnp.float32),
            pltpu.SemaphoreType.DMA,
            pltpu.SemaphoreType.DMA,
            pltpu.SemaphoreType.DMA,
            pltpu.SemaphoreType.DMA,
        ],
    )(a_flat, u_flat, src, dst, w, w_readout)


def _tc_pre_kernel(x_ref, h0_ref, h1_ref, b_ref, u_ref, a_ref):
    x = x_ref[...]
    u_ref[...] = jnp.dot(h0_ref[...], x,
                         preferred_element_type=jnp.float32) + b_ref[...]
    a_ref[...] = jnp.dot(h1_ref[...], x, preferred_element_type=jnp.float32)


def _tc_pre(x, h0, h1, b):
    return pl.pallas_call(
        _tc_pre_kernel,
        out_shape=[
            jax.ShapeDtypeStruct((F, N), jnp.float32),
            jax.ShapeDtypeStruct((F, N), jnp.float32),
        ],
    )(x, h0, h1, b)


def kernel(x, edge_index_clique, edge_weight_clique, edge_index_line,
           edge_weight_line, h_clique, b_clique, h_line, b_line, W_mlp, b_mlp):
    X = x[0]

    h0c = h_clique[:, 0, 0, :]
    h1c = h_clique[:, 0, 1, :]
    h0l = h_line[:, 0, 0, :]
    h1l = h_line[:, 0, 1, :]

    src_c = edge_index_clique[1].astype(jnp.int32)
    dst_c = edge_index_clique[0].astype(jnp.int32)
    src_l = edge_index_line[1].astype(jnp.int32)
    dst_l = edge_index_line[0].astype(jnp.int32)
    w_flat = W_mlp.reshape(F * N)

    u1, a1 = _tc_pre(X, h0c, h1c, b_clique)
    y1 = _spmm_sc(False, a1.reshape(F * N), u1.reshape(F * N),
                  src_c, dst_c, edge_weight_clique, w_flat)
    u2, a2 = _tc_pre(y1.reshape(F, N), h0l, h1l, b_line)
    parts = _spmm_sc(True, a2.reshape(F * N), u2.reshape(F * N),
                     src_l, dst_l, edge_weight_line, w_flat)
    return (jnp.sum(parts) + b_mlp[0]).reshape(1, 1)
